# baseline (speedup 1.0000x reference)
import jax
import jax.numpy as jnp
from jax import lax
from jax.experimental import pallas as pl
from jax.experimental.pallas import tpu as pltpu
from jax.experimental.pallas import tpu_sc as plsc

N_ROWS = 1_000_000
D = 32
N_UPD = 16_384
NC = 2
NS = 16
NW = NC * NS
SC_COLS = 999_936
TAILC = N_ROWS - SC_COLS
L = 16
CB = 1024
NCH_A = 31
NCH_B = 30
XBASE = NCH_B * CB
XTRA = 512
UTAB = NCH_A * CB
SB = 16
NSB = CB // SB
VROW = 128


def _neg_pad_body(a0t_ref, o_ref):
    x = a0t_ref[...]
    o_ref[:, 0:D] = -jnp.transpose(x)
    o_ref[:, D:VROW] = jnp.zeros((x.shape[1], VROW - D), jnp.float32)


def _neg_pad(a0t):
    blk = 2048
    return pl.pallas_call(
        _neg_pad_body,
        out_shape=jax.ShapeDtypeStruct((N_UPD, VROW), jnp.float32),
        grid=(N_UPD // blk,),
        in_specs=[pl.BlockSpec((D, blk), lambda i: (0, i))],
        out_specs=pl.BlockSpec((blk, VROW), lambda i: (i, 0)),
    )(a0t)


def _tail_patch_body(out_ref, a1_ref, a0t_ref, idx_ref, o_ref):
    del out_ref
    acc = a1_ref[...]
    idxg = idx_ref[...]
    posg = (lax.broadcasted_iota(jnp.int32, (128, 128), 0) * 128
            + lax.broadcasted_iota(jnp.int32, (128, 128), 1))
    cvec = lax.broadcasted_iota(jnp.int32, (1, 128), 1)
    wpv = jnp.full((1, 128), -1, jnp.int32)
    for c in range(TAILC):
        sel = jnp.where(idxg == SC_COLS + c, posg, -1)
        wp = jnp.max(sel)
        wpv = jnp.where(cvec == c, wp, wpv)
    valid = wpv >= 0
    sel_mat = (lax.broadcasted_iota(jnp.int32, (N_UPD, 128), 0)
               == jnp.broadcast_to(wpv, (N_UPD, 128))).astype(jnp.float32)
    vals = lax.dot_general(a0t_ref[...], sel_mat, (((1,), (0,)), ((), ())),
                           preferred_element_type=jnp.float32)
    o_ref[...] = jnp.where(jnp.broadcast_to(valid, (D, 128)), -vals, acc)


def _tail_patch(out_t, a1t, a0t, idx):
    idxg = jnp.reshape(idx, (128, 128))
    tb = SC_COLS // 128
    return pl.pallas_call(
        _tail_patch_body,
        out_shape=jax.ShapeDtypeStruct((D, N_ROWS), jnp.float32),
        grid=(1,),
        in_specs=[
            pl.BlockSpec(memory_space=pl.ANY),
            pl.BlockSpec((D, 128), lambda i: (0, tb)),
            pl.BlockSpec((D, N_UPD), lambda i: (0, 0)),
            pl.BlockSpec((128, 128), lambda i: (0, 0)),
        ],
        out_specs=pl.BlockSpec((D, 128), lambda i: (0, tb)),
        input_output_aliases={0: 0},
    )(out_t, a1t, a0t, idxg)


def _sc_merge_kernel(neg0p_hbm, a1t_hbm, idx_hbm, out_hbm,
                     idx_v, utab,
                     buf0, buf1, cp0, cp1, cc0, cc1, vals0, vals1,
                     in_s0, in_s1, out_s0, out_s1, gat_s0, gat_s1, gat2_s):
    bufs = (buf0, buf1)
    cps = (cp0, cp1)
    ccs = (cc0, cc1)
    valss = (vals0, vals1)
    in_s = (in_s0, in_s1)
    out_s = (out_s0, out_s1)
    gat_s = (gat_s0, gat_s1)

    wid = lax.axis_index("s") * NC + lax.axis_index("c")
    is_a = wid < 16
    is_last = wid == NW - 1
    lo = pl.multiple_of(
        jnp.where(is_a, wid * (NCH_A * CB),
                  16 * NCH_A * CB + (wid - 16) * (NCH_B * CB)), 128)
    nch = jnp.where(is_a, NCH_A, NCH_B)
    hi = lo + nch * CB + jnp.where(is_last, XTRA, 0)

    def col0(k):
        return pl.multiple_of(lo + k * CB, 128)

    pltpu.async_copy(a1t_hbm.at[:, pl.ds(col0(0), CB)], bufs[0], in_s[0])

    lane = lax.iota(jnp.int32, L)
    neg1 = jnp.full((L,), -1, dtype=jnp.int32)

    def init_body(i, _):
        utab[pl.ds(i * L, L)] = neg1
        return 0

    lax.fori_loop(0, UTAB // L, init_body, 0, unroll=8)

    for half in range(2):
        pltpu.sync_copy(idx_hbm.at[pl.ds(half * (N_UPD // 2), N_UPD // 2)],
                        idx_v)

        def filt_body(g, _):
            v = idx_v[pl.ds(g * L, L)]
            m = (v >= lo) & (v < hi)
            pos = half * (N_UPD // 2) + g * L + lane
            plsc.store_scatter(utab, [v - lo], pos, mask=m)
            return 0

        lax.fori_loop(0, N_UPD // 2 // L, filt_body, 0, unroll=4)

    def seed_body(i, _):
        rvec = jnp.full((L,), lax.shift_right_logical(i, 3), jnp.int32)
        cvec = jnp.bitwise_and(i, 7) * L + lane
        pad = jnp.bitwise_and(wid * (NSB * SB) + i * L + lane, N_UPD - 1)
        plsc.store_scatter(cp0, [rvec, cvec], pad)
        plsc.store_scatter(cp1, [rvec, cvec], pad)
        return 0

    lax.fori_loop(0, NSB, seed_body, 0, unroll=4)

    def scan_chunk(k, par):
        base = jnp.minimum(k * CB, UTAB - CB)
        del k

        def scan_body(g, ucnt):
            wp = utab[pl.ds(base + g * L, L)]
            m = wp >= 0
            npop = plsc.all_reduce_population_count(m)[0]

            @pl.when(npop > 0)
            def _():
                mi = m.astype(jnp.int32)
                pref = plsc.cumsum(mi)
                t = ucnt + pref - 1
                trow = lax.shift_right_logical(jnp.maximum(t, 0), 7)
                tcol = jnp.bitwise_and(t, 127)
                plsc.store_scatter(cps[par], [trow, tcol], wp, mask=m)
                plsc.store_scatter(ccs[par], [trow, tcol], g * L + lane,
                                   mask=m)

            return ucnt + npop

        return lax.fori_loop(0, CB // L, scan_body, jnp.int32(0), unroll=8)

    def start_gather(par):
        pltpu.async_copy(neg0p_hbm.at[cps[par].at[0, pl.ds(0, SB)]],
                         valss[par], gat_s[par])

    def apply_chunk(par, ucnt):
        @pl.when(ucnt > 0)
        def _():
            pltpu.make_async_copy(
                neg0p_hbm.at[pl.ds(0, SB)], valss[par], gat_s[par]).wait()

            def batch_body(b2, _):
                brow = lax.shift_right_logical(b2, 3)
                bcol = jnp.bitwise_and(b2, 7) * L

                @pl.when(b2 > 0)
                def _():
                    pltpu.async_copy(
                        neg0p_hbm.at[cps[par].at[brow, pl.ds(bcol, SB)]],
                        valss[par], gat2_s).wait()

                j = lane
                valid = (b2 * SB + j) < ucnt
                browv = jnp.full((L,), brow, jnp.int32)
                ccol = plsc.load_gather(ccs[par], [browv, bcol + j],
                                        mask=valid)
                for r in range(D):
                    rvec = jnp.full((L,), r, jnp.int32)
                    x = plsc.load_gather(valss[par], [j, rvec], mask=valid)
                    plsc.store_scatter(bufs[par], [rvec, ccol], x,
                                       mask=valid)
                return 0

            nb = lax.div(ucnt + SB - 1, jnp.int32(SB))
            lax.fori_loop(0, nb, batch_body, 0, unroll=False)

    ucnt0 = scan_chunk(jnp.int32(0), 0)


    @pl.when(ucnt0 > 0)
    def _():
        start_gather(0)

    def slot(k, par, ucnt_in):
        live = k < nch

        @pl.when(live)
        def _():
            pltpu.make_async_copy(
                a1t_hbm.at[:, pl.ds(col0(k), CB)], bufs[par],
                in_s[par]).wait()
            @pl.when(k >= 1)
            def _():
                pltpu.make_async_copy(
                    bufs[1 - par], out_hbm.at[:, pl.ds(col0(k), CB)],
                    out_s[1 - par]).wait()

            @pl.when(k + 1 < nch)
            def _():
                pltpu.async_copy(a1t_hbm.at[:, pl.ds(col0(k + 1), CB)],
                                 bufs[1 - par], in_s[1 - par])

            apply_chunk(par, ucnt_in)
            pltpu.async_copy(bufs[par],
                             out_hbm.at[:, pl.ds(col0(k), CB)], out_s[par])

        ucnt_next = scan_chunk(jnp.minimum(k + 1, NCH_A - 1), 1 - par)

        @pl.when(live & (k + 1 < nch) & (ucnt_next > 0))
        def _():
            start_gather(1 - par)

        return ucnt_next

    def pair_body(p, ucnt):
        ucnt = slot(2 * p, 0, ucnt)
        ucnt = slot(2 * p + 1, 1, ucnt)
        return ucnt

    lax.fori_loop(0, (NCH_A + 1) // 2, pair_body, ucnt0, unroll=False)

    @pl.when(is_a)
    def _():
        pltpu.make_async_copy(
            bufs[0], out_hbm.at[:, pl.ds(col0(0), CB)], out_s[0]).wait()

    @pl.when(jnp.logical_not(is_a))
    def _():
        pltpu.make_async_copy(
            bufs[1], out_hbm.at[:, pl.ds(col0(0), CB)], out_s[1]).wait()

    @pl.when(is_last)
    def _():
        xc = pl.multiple_of(lo + XBASE, 128)
        pltpu.sync_copy(a1t_hbm.at[:, pl.ds(xc, XTRA)],
                        bufs[0].at[:, pl.ds(0, XTRA)])
        xcnt = scan_chunk(jnp.int32(NCH_B), 0)

        @pl.when(xcnt > 0)
        def _():
            start_gather(0)

        apply_chunk(0, xcnt)
        pltpu.sync_copy(bufs[0].at[:, pl.ds(0, XTRA)],
                        out_hbm.at[:, pl.ds(xc, XTRA)])


@jax.jit
def _scatter_overwrite(arg0, arg1, idx):
    a0t = jnp.transpose(arg0)
    a1t = jnp.transpose(arg1)
    neg0p = _neg_pad(a0t)
    mesh = plsc.VectorSubcoreMesh(
        core_axis_name="c", subcore_axis_name="s",
        num_cores=NC, num_subcores=NS)
    f = pl.kernel(
        _sc_merge_kernel,
        out_type=jax.ShapeDtypeStruct((D, N_ROWS), jnp.float32),
        mesh=mesh,
        compiler_params=pltpu.CompilerParams(needs_layout_passes=False),
        scratch_types=[
            pltpu.VMEM((N_UPD // 2,), jnp.int32),
            pltpu.VMEM((UTAB,), jnp.int32),
            pltpu.VMEM((D, CB), jnp.float32),
            pltpu.VMEM((D, CB), jnp.float32),
            pltpu.VMEM((NSB // 8, 128), jnp.int32),
            pltpu.VMEM((NSB // 8, 128), jnp.int32),
            pltpu.VMEM((NSB // 8, 128), jnp.int32),
            pltpu.VMEM((NSB // 8, 128), jnp.int32),
            pltpu.VMEM((SB, VROW), jnp.float32),
            pltpu.VMEM((SB, VROW), jnp.float32),
            pltpu.SemaphoreType.DMA,
            pltpu.SemaphoreType.DMA,
            pltpu.SemaphoreType.DMA,
            pltpu.SemaphoreType.DMA,
            pltpu.SemaphoreType.DMA,
            pltpu.SemaphoreType.DMA,
            pltpu.SemaphoreType.DMA,
        ],
    )
    out_t = f(neg0p, a1t, idx)
    out_t = _tail_patch(out_t, a1t, a0t, idx)
    return jnp.transpose(out_t)


def kernel(arg0_1, arg1_1, arg2_1):
    idx = arg2_1.astype(jnp.int32)
    return (_scatter_overwrite(arg0_1, arg1_1, idx),)


# speedup vs baseline: 1.1070x; 1.1070x over previous
import jax
import jax.numpy as jnp
from jax import lax
from jax.experimental import pallas as pl
from jax.experimental.pallas import tpu as pltpu
from jax.experimental.pallas import tpu_sc as plsc

N_ROWS = 1_000_000
D = 32
N_UPD = 16_384
NC = 2
NS = 16
NW = NC * NS
SC_COLS = 999_936
TAILC = N_ROWS - SC_COLS
L = 16
CB = 1024
NCH_A = 31
NCH_B = 30
XBASE = NCH_B * CB
XTRA = 512
UTAB = NCH_A * CB
SB = 16
NSB = CB // SB
VROW = 128


def _neg_pad_body(a0t_ref, o_ref):
    x = a0t_ref[...]
    o_ref[:, 0:D] = -jnp.transpose(x)
    o_ref[:, D:VROW] = jnp.zeros((x.shape[1], VROW - D), jnp.float32)


def _neg_pad(a0t):
    blk = 2048
    return pl.pallas_call(
        _neg_pad_body,
        out_shape=jax.ShapeDtypeStruct((N_UPD, VROW), jnp.float32),
        grid=(N_UPD // blk,),
        in_specs=[pl.BlockSpec((D, blk), lambda i: (0, i))],
        out_specs=pl.BlockSpec((blk, VROW), lambda i: (i, 0)),
    )(a0t)


def _tail_patch_body(out_ref, a1_ref, a0t_ref, idx_ref, o_ref):
    del out_ref
    acc = a1_ref[...]
    idxg = idx_ref[...]
    posg = (lax.broadcasted_iota(jnp.int32, (128, 128), 0) * 128
            + lax.broadcasted_iota(jnp.int32, (128, 128), 1))
    cvec = lax.broadcasted_iota(jnp.int32, (1, 128), 1)
    wpv = jnp.full((1, 128), -1, jnp.int32)
    for c in range(TAILC):
        sel = jnp.where(idxg == SC_COLS + c, posg, -1)
        wp = jnp.max(sel)
        wpv = jnp.where(cvec == c, wp, wpv)
    valid = wpv >= 0
    sel_mat = (lax.broadcasted_iota(jnp.int32, (N_UPD, 128), 0)
               == jnp.broadcast_to(wpv, (N_UPD, 128))).astype(jnp.float32)
    vals = lax.dot_general(a0t_ref[...], sel_mat, (((1,), (0,)), ((), ())),
                           preferred_element_type=jnp.float32)
    o_ref[...] = jnp.where(jnp.broadcast_to(valid, (D, 128)), -vals, acc)


def _tail_patch(out_t, a1t, a0t, idx):
    idxg = jnp.reshape(idx, (128, 128))
    tb = SC_COLS // 128
    return pl.pallas_call(
        _tail_patch_body,
        out_shape=jax.ShapeDtypeStruct((D, N_ROWS), jnp.float32),
        grid=(1,),
        in_specs=[
            pl.BlockSpec(memory_space=pl.ANY),
            pl.BlockSpec((D, 128), lambda i: (0, tb)),
            pl.BlockSpec((D, N_UPD), lambda i: (0, 0)),
            pl.BlockSpec((128, 128), lambda i: (0, 0)),
        ],
        out_specs=pl.BlockSpec((D, 128), lambda i: (0, tb)),
        input_output_aliases={0: 0},
    )(out_t, a1t, a0t, idxg)


def _sc_merge_kernel(neg0p_hbm, a1t_hbm, idx_hbm, out_hbm,
                     idx_v, utab,
                     buf0, buf1, cp0, cp1, cc0, cc1, vals0, vals1,
                     in_s0, in_s1, out_s0, out_s1, gat_s0, gat_s1, gat2_s):
    bufs = (buf0, buf1)
    cps = (cp0, cp1)
    ccs = (cc0, cc1)
    valss = (vals0, vals1)
    in_s = (in_s0, in_s1)
    out_s = (out_s0, out_s1)
    gat_s = (gat_s0, gat_s1)

    wid = lax.axis_index("s") * NC + lax.axis_index("c")
    is_a = wid < 16
    is_last = wid == NW - 1
    lo = pl.multiple_of(
        jnp.where(is_a, wid * (NCH_A * CB),
                  16 * NCH_A * CB + (wid - 16) * (NCH_B * CB)), 128)
    nch = jnp.where(is_a, NCH_A, NCH_B)
    hi = lo + nch * CB + jnp.where(is_last, XTRA, 0)

    def col0(k):
        return pl.multiple_of(lo + k * CB, 128)

    pltpu.async_copy(a1t_hbm.at[:, pl.ds(col0(0), CB)], bufs[0], in_s[0])

    lane = lax.iota(jnp.int32, L)
    neg1 = jnp.full((L,), -1, dtype=jnp.int32)

    def init_body(i, _):
        utab[pl.ds(i * L, L)] = neg1
        return 0

    lax.fori_loop(0, UTAB // L, init_body, 0, unroll=8)

    for half in range(2):
        pltpu.sync_copy(idx_hbm.at[pl.ds(half * (N_UPD // 2), N_UPD // 2)],
                        idx_v)

        def filt_body(g, _):
            v = idx_v[pl.ds(g * L, L)]
            m = (v >= lo) & (v < hi)
            pos = half * (N_UPD // 2) + g * L + lane
            plsc.store_scatter(utab, [v - lo], pos, mask=m)
            return 0

        lax.fori_loop(0, N_UPD // 2 // L, filt_body, 0, unroll=4)

    def seed_body(i, _):
        rvec = jnp.full((L,), lax.shift_right_logical(i, 3), jnp.int32)
        cvec = jnp.bitwise_and(i, 7) * L + lane
        pad = jnp.bitwise_and(wid * (NSB * SB) + i * L + lane, N_UPD - 1)
        plsc.store_scatter(cp0, [rvec, cvec], pad)
        plsc.store_scatter(cp1, [rvec, cvec], pad)
        return 0

    lax.fori_loop(0, NSB, seed_body, 0, unroll=4)

    def scan_chunk(k, par):
        base = jnp.minimum(k * CB, UTAB - CB)
        del k

        def scan_body(g, ucnt):
            wp = utab[pl.ds(base + g * L, L)]
            m = wp >= 0
            mi = m.astype(jnp.int32)
            pref = plsc.cumsum(mi)
            t = ucnt + pref - 1
            trow = lax.shift_right_logical(jnp.maximum(t, 0), 7)
            tcol = jnp.bitwise_and(t, 127)
            plsc.store_scatter(cps[par], [trow, tcol], wp, mask=m)
            plsc.store_scatter(ccs[par], [trow, tcol], g * L + lane, mask=m)
            return ucnt + plsc.all_reduce_population_count(m)[0]

        return lax.fori_loop(0, CB // L, scan_body, jnp.int32(0), unroll=4)

    def start_gather(par):
        pltpu.async_copy(neg0p_hbm.at[cps[par].at[0, pl.ds(0, SB)]],
                         valss[par], gat_s[par])

    def apply_chunk(par, ucnt):
        @pl.when(ucnt > 0)
        def _():
            pltpu.make_async_copy(
                neg0p_hbm.at[pl.ds(0, SB)], valss[par], gat_s[par]).wait()

            def batch_body(b2, _):
                brow = lax.shift_right_logical(b2, 3)
                bcol = jnp.bitwise_and(b2, 7) * L

                @pl.when(b2 > 0)
                def _():
                    pltpu.async_copy(
                        neg0p_hbm.at[cps[par].at[brow, pl.ds(bcol, SB)]],
                        valss[par], gat2_s).wait()

                j = lane
                valid = (b2 * SB + j) < ucnt
                browv = jnp.full((L,), brow, jnp.int32)
                ccol = plsc.load_gather(ccs[par], [browv, bcol + j],
                                        mask=valid)
                for r in range(D):
                    rvec = jnp.full((L,), r, jnp.int32)
                    x = plsc.load_gather(valss[par], [j, rvec], mask=valid)
                    plsc.store_scatter(bufs[par], [rvec, ccol], x,
                                       mask=valid)
                return 0

            nb = lax.div(ucnt + SB - 1, jnp.int32(SB))
            lax.fori_loop(0, nb, batch_body, 0, unroll=False)

    ucnt0 = scan_chunk(jnp.int32(0), 0)


    @pl.when(ucnt0 > 0)
    def _():
        start_gather(0)

    def slot(k, par, ucnt_in):
        live = k < nch

        @pl.when(live)
        def _():
            pltpu.make_async_copy(
                a1t_hbm.at[:, pl.ds(col0(k), CB)], bufs[par],
                in_s[par]).wait()
            @pl.when(k >= 1)
            def _():
                pltpu.make_async_copy(
                    bufs[1 - par], out_hbm.at[:, pl.ds(col0(k), CB)],
                    out_s[1 - par]).wait()

            @pl.when(k + 1 < nch)
            def _():
                pltpu.async_copy(a1t_hbm.at[:, pl.ds(col0(k + 1), CB)],
                                 bufs[1 - par], in_s[1 - par])

            apply_chunk(par, ucnt_in)
            pltpu.async_copy(bufs[par],
                             out_hbm.at[:, pl.ds(col0(k), CB)], out_s[par])

        ucnt_next = scan_chunk(jnp.minimum(k + 1, NCH_A - 1), 1 - par)

        @pl.when(live & (k + 1 < nch) & (ucnt_next > 0))
        def _():
            start_gather(1 - par)

        return ucnt_next

    def pair_body(p, ucnt):
        ucnt = slot(2 * p, 0, ucnt)
        ucnt = slot(2 * p + 1, 1, ucnt)
        return ucnt

    lax.fori_loop(0, (NCH_A + 1) // 2, pair_body, ucnt0, unroll=False)

    @pl.when(is_a)
    def _():
        pltpu.make_async_copy(
            bufs[0], out_hbm.at[:, pl.ds(col0(0), CB)], out_s[0]).wait()

    @pl.when(jnp.logical_not(is_a))
    def _():
        pltpu.make_async_copy(
            bufs[1], out_hbm.at[:, pl.ds(col0(0), CB)], out_s[1]).wait()

    @pl.when(is_last)
    def _():
        xc = pl.multiple_of(lo + XBASE, 128)
        pltpu.sync_copy(a1t_hbm.at[:, pl.ds(xc, XTRA)],
                        bufs[0].at[:, pl.ds(0, XTRA)])
        xcnt = scan_chunk(jnp.int32(NCH_B), 0)

        @pl.when(xcnt > 0)
        def _():
            start_gather(0)

        apply_chunk(0, xcnt)
        pltpu.sync_copy(bufs[0].at[:, pl.ds(0, XTRA)],
                        out_hbm.at[:, pl.ds(xc, XTRA)])


@jax.jit
def _scatter_overwrite(arg0, arg1, idx):
    a0t = jnp.transpose(arg0)
    a1t = jnp.transpose(arg1)
    neg0p = _neg_pad(a0t)
    mesh = plsc.VectorSubcoreMesh(
        core_axis_name="c", subcore_axis_name="s",
        num_cores=NC, num_subcores=NS)
    f = pl.kernel(
        _sc_merge_kernel,
        out_type=jax.ShapeDtypeStruct((D, N_ROWS), jnp.float32),
        mesh=mesh,
        compiler_params=pltpu.CompilerParams(needs_layout_passes=False),
        scratch_types=[
            pltpu.VMEM((N_UPD // 2,), jnp.int32),
            pltpu.VMEM((UTAB,), jnp.int32),
            pltpu.VMEM((D, CB), jnp.float32),
            pltpu.VMEM((D, CB), jnp.float32),
            pltpu.VMEM((NSB // 8, 128), jnp.int32),
            pltpu.VMEM((NSB // 8, 128), jnp.int32),
            pltpu.VMEM((NSB // 8, 128), jnp.int32),
            pltpu.VMEM((NSB // 8, 128), jnp.int32),
            pltpu.VMEM((SB, VROW), jnp.float32),
            pltpu.VMEM((SB, VROW), jnp.float32),
            pltpu.SemaphoreType.DMA,
            pltpu.SemaphoreType.DMA,
            pltpu.SemaphoreType.DMA,
            pltpu.SemaphoreType.DMA,
            pltpu.SemaphoreType.DMA,
            pltpu.SemaphoreType.DMA,
            pltpu.SemaphoreType.DMA,
        ],
    )
    out_t = f(neg0p, a1t, idx)
    out_t = _tail_patch(out_t, a1t, a0t, idx)
    return jnp.transpose(out_t)


def kernel(arg0_1, arg1_1, arg2_1):
    idx = arg2_1.astype(jnp.int32)
    return (_scatter_overwrite(arg0_1, arg1_1, idx),)
